# baseline (speedup 1.0000x reference)
import jax
import jax.numpy as jnp
from jax.experimental import pallas as pl
from jax.experimental.pallas import tpu as pltpu


def _in_copy(x_hbm, ob, insems, b, buf, D):
    return pltpu.make_async_copy(
        x_hbm.at[b], ob.at[buf, :, pl.ds(0, D)], insems.at[buf])


def _out_copy(o_hbm, ob, outsems, b, buf):
    return pltpu.make_async_copy(ob.at[buf], o_hbm.at[b], outsems.at[buf])


def _body(x_hbm, mf_hbm, o_hbm, ob, mv, insems, outsems, msem):
    B, S, D = x_hbm.shape

    mcp = pltpu.make_async_copy(mf_hbm, mv, msem)
    mcp.start()
    _in_copy(x_hbm, ob, insems, 0, 0, D).start()
    _in_copy(x_hbm, ob, insems, 1, 1, D).start()
    mcp.wait()

    def step(b, _):
        buf = jax.lax.rem(b, 3)
        _in_copy(x_hbm, ob, insems, b, buf, D).wait()

        x = ob[buf, :, pl.ds(0, D)]
        m1 = mv[b]
        s = jax.lax.dot_general(
            m1, x, (((1,), (0,)), ((), ())),
            preferred_element_type=jnp.float32,
            precision=jax.lax.Precision.DEFAULT)
        cnt = jnp.sum(m1)
        mean = s / cnt
        ob[buf, :, pl.ds(D, D)] = jnp.broadcast_to(mean, (S, D))

        @pl.when(b >= 1)
        def _():
            _out_copy(o_hbm, ob, outsems, b - 1, jax.lax.rem(b + 2, 3)).wait()

        _out_copy(o_hbm, ob, outsems, b, buf).start()

        @pl.when(b + 2 < B)
        def _():
            _in_copy(x_hbm, ob, insems, b + 2, jax.lax.rem(b + 2, 3), D).start()
        return 0

    jax.lax.fori_loop(0, B, step, 0)
    _out_copy(o_hbm, ob, outsems, B - 1, jax.lax.rem(B - 1, 3)).wait()


def kernel(inputs, mask):
    B, S, D = inputs.shape
    mf = mask.astype(inputs.dtype).reshape(B, 1, S)

    out = pl.pallas_call(
        _body,
        in_specs=[
            pl.BlockSpec(memory_space=pltpu.HBM),
            pl.BlockSpec(memory_space=pltpu.HBM),
        ],
        out_specs=pl.BlockSpec(memory_space=pltpu.HBM),
        out_shape=jax.ShapeDtypeStruct((B, S, 2 * D), inputs.dtype),
        scratch_shapes=[
            pltpu.VMEM((3, S, 2 * D), inputs.dtype),
            pltpu.VMEM((B, 1, S), inputs.dtype),
            pltpu.SemaphoreType.DMA((3,)),
            pltpu.SemaphoreType.DMA((3,)),
            pltpu.SemaphoreType.DMA,
        ],
        compiler_params=pltpu.CompilerParams(
            vmem_limit_bytes=60 * 1024 * 1024,
        ),
    )(inputs, mf)
    return out
